# baseline (speedup 1.0000x reference)
import functools

import jax
import jax.numpy as jnp
from jax import lax
from jax.experimental import pallas as pl
from jax.experimental.pallas import tpu as pltpu
from jax.experimental.pallas import tpu_sc as plsc

B = 4096
D = 2048
E = 16
H = 512
T = 2
K = 2

BLK = 256
R = T * B * K
XR = R + E * BLK
G = XR // BLK

_NC = 2
_NS = 16
_NW = _NC * _NS


def _gate_body(x_ref, wcat_ref, eps_ref, meta_ref, xpk_ref):
    x = x_ref[...]
    r = lax.bitcast_convert_type(x, jnp.int32)
    rb = r + 0x7FFF + ((r >> 16) & 1)
    hi = rb[:, :D // 2] & jnp.int32(-65536)
    lo = lax.shift_right_logical(rb[:, D // 2:], 16)
    xpk_ref[...] = hi | lo
    proj = jnp.dot(x, wcat_ref[...], preferred_element_type=jnp.float32)
    ii = jax.lax.broadcasted_iota(jnp.int32, (x.shape[0], E), 1)
    lane8 = jax.lax.broadcasted_iota(jnp.int32, (x.shape[0], 8), 1)
    for t in range(T):
        mean = proj[:, t * E:(t + 1) * E]
        npj = proj[:, (T + t) * E:(T + t + 1) * E]
        std = jnp.maximum(npj, 0.0) + jnp.log1p(jnp.exp(-jnp.abs(npj)))
        noisy = mean + eps_ref[t] * std
        v1 = jnp.max(noisy, axis=1, keepdims=True)
        first1 = jnp.min(jnp.where(noisy == v1, ii, E), axis=1, keepdims=True)
        n2 = jnp.where(ii == first1, -jnp.inf, noisy)
        v2 = jnp.max(n2, axis=1, keepdims=True)
        first2 = jnp.min(jnp.where(n2 == v2, ii, E), axis=1, keepdims=True)
        z = jnp.exp(v2 - v1)
        w1 = 1.0 / (1.0 + z)
        w2 = 1.0 - w1
        m = jnp.where(lane8 == 0, first1.astype(jnp.float32),
            jnp.where(lane8 == 1, first2.astype(jnp.float32),
            jnp.where(lane8 == 2, w1, jnp.where(lane8 == 3, w2, 0.0))))
        meta_ref[t] = m


def _grouped_body(be_s, xg_ref, we0_ref, be0_ref, we1_ref, be1_ref, out_ref):
    u = xg_ref[...]
    xa = lax.bitcast_convert_type(
        u & jnp.int32(-65536), jnp.float32).astype(jnp.bfloat16)
    xb = lax.bitcast_convert_type(
        lax.shift_left(u, 16), jnp.float32).astype(jnp.bfloat16)
    w0 = we0_ref[0].astype(jnp.bfloat16)
    acc = (jnp.dot(xa, w0[:D // 2], preferred_element_type=jnp.float32)
           + jnp.dot(xb, w0[D // 2:], preferred_element_type=jnp.float32))
    h = jnp.maximum(acc + be0_ref[0], 0.0).astype(jnp.bfloat16)
    w1 = we1_ref[0].astype(jnp.bfloat16)
    o = jnp.dot(h, w1, preferred_element_type=jnp.float32) + be1_ref[0]
    out_ref[...] = o


def _grouped_body_alias(be_s, xg_ref, we0_ref, be0_ref, we1_ref, be1_ref,
                        prev_ref, out_ref):
    _grouped_body(be_s, xg_ref, we0_ref, be0_ref, we1_ref, be1_ref, out_ref)


def _heads_body(rc_ref, w0_ref, w1_ref, wh0_ref, bh0_ref, wh1_ref, bh1_ref,
                wh2_ref, bh2_ref, out_ref):
    go = w0_ref[0] * rc_ref[0, 0] + w1_ref[0] * rc_ref[1, 0]
    a = jnp.maximum(jnp.dot(go, wh0_ref[0],
                            preferred_element_type=jnp.float32) + bh0_ref[0],
                    0.0)
    b = jnp.maximum(jnp.dot(a, wh1_ref[0],
                            preferred_element_type=jnp.float32) + bh1_ref[0],
                    0.0)
    out_ref[0] = jnp.dot(b, wh2_ref[0],
                         preferred_element_type=jnp.float32) + bh2_ref[0]


def _sc_gather(table, idx, n_rows, chunk):
    dd = table.shape[1]
    per_w = n_rows // _NW
    n_chunks = per_w // chunk
    assert per_w % chunk == 0
    nb = 3
    n_main = (n_chunks // nb) * nb
    mesh = plsc.VectorSubcoreMesh(core_axis_name="c", subcore_axis_name="s")

    @functools.partial(
        pl.kernel, mesh=mesh,
        out_type=jax.ShapeDtypeStruct((n_rows, dd), table.dtype),
        scratch_types=(
            [pltpu.VMEM((chunk,), jnp.int32) for _ in range(nb)]
            + [pltpu.VMEM((chunk, dd), table.dtype) for _ in range(nb)]
            + [pltpu.SemaphoreType.DMA, pltpu.SemaphoreType.DMA]
        ),
    )
    def k(table_hbm, idx_hbm, out_hbm, i0, i1, i2, b0, b1, b2, gsem, osem):
        idxs = [i0, i1, i2]
        bufs = [b0, b1, b2]
        wid = lax.axis_index("s") * _NC + lax.axis_index("c")
        base = wid * per_w

        def drain(l, drain_i):
            pltpu.make_async_copy(
                bufs[l],
                out_hbm.at[pl.ds(base + drain_i * chunk, chunk)],
                osem).wait()

        def chunk_step(i, l, drain_i, guard):
            if guard:
                @pl.when(drain_i >= 0)
                def _():
                    drain(l, drain_i)
            elif drain_i >= 0:
                drain(l, drain_i)
            pltpu.sync_copy(idx_hbm.at[pl.ds(base + i * chunk, chunk)],
                            idxs[l])
            pltpu.async_copy(table_hbm.at[idxs[l]], bufs[l], gsem).wait()
            pltpu.async_copy(
                bufs[l], out_hbm.at[pl.ds(base + i * chunk, chunk)], osem)

        def body(j, carry):
            for l in range(nb):
                i = j * nb + l
                chunk_step(i, l, i - nb, True)
            return carry

        lax.fori_loop(0, n_main // nb, body, 0)
        for i in range(n_main, n_chunks):
            chunk_step(i, i % nb, i - nb, False)
        for i in range(max(0, n_chunks - nb), n_chunks):
            pltpu.make_async_copy(
                bufs[i % nb],
                out_hbm.at[pl.ds(base + i * chunk, chunk)], osem).wait()

    return k(table, idx)


def kernel(x, We0, be0, We1, be1, Wg, Wn, Wh0, bh0, Wh1, bh1, Wh2, bh2):
    eps_key = jax.random.key(42)
    eps = jnp.stack([
        jax.random.normal(jax.random.fold_in(eps_key, i), (B, E), jnp.float32)
        for i in range(T)])
    wcat = jnp.concatenate([Wg[0], Wg[1], Wn[0], Wn[1]], axis=1)

    GB = 1024
    meta = pl.pallas_call(
        _gate_body,
        grid=(B // GB,),
        in_specs=[
            pl.BlockSpec((GB, D), lambda i: (i, 0)),
            pl.BlockSpec((D, 4 * E), lambda i: (0, 0)),
            pl.BlockSpec((T, GB, E), lambda i: (0, i, 0)),
        ],
        out_specs=[
            pl.BlockSpec((T, GB, 8), lambda i: (0, i, 0)),
            pl.BlockSpec((GB, D // 2), lambda i: (i, 0)),
        ],
        out_shape=[
            jax.ShapeDtypeStruct((T, B, 8), jnp.float32),
            jax.ShapeDtypeStruct((B, D // 2), jnp.int32),
        ],
    )(x, wcat, eps)
    meta, x_pk = meta

    idx = meta[:, :, 0:2].astype(jnp.int32)
    e_flat = idx.reshape(-1)
    tok_flat = jnp.broadcast_to(jnp.arange(B)[None, :, None],
                                (T, B, K)).reshape(-1)
    oh = (e_flat[:, None] == jnp.arange(E)[None, :]).astype(jnp.int32)
    counts = jnp.sum(oh, axis=0)
    rank = jnp.take_along_axis(jnp.cumsum(oh, axis=0), e_flat[:, None],
                               axis=1)[:, 0] - 1
    P = ((counts + BLK - 1) // BLK) * BLK
    cp = jnp.cumsum(P)
    poff = cp - P
    pos = poff[e_flat] + rank
    row_token = jnp.zeros((XR,), jnp.int32).at[pos].set(tok_flat)
    block_expert = jnp.minimum(
        jnp.searchsorted(cp // BLK, jnp.arange(G), side='right'),
        E - 1).astype(jnp.int32)

    xg = _sc_gather(x_pk, row_token, XR, 32)

    out_rows = pl.pallas_call(
        _grouped_body,
        grid_spec=pltpu.PrefetchScalarGridSpec(
            num_scalar_prefetch=1,
            grid=(G,),
            in_specs=[
                pl.BlockSpec((BLK, D // 2), lambda g, be: (g, 0)),
                pl.BlockSpec((1, D, H), lambda g, be: (be[g], 0, 0)),
                pl.BlockSpec((1, 1, H), lambda g, be: (be[g], 0, 0)),
                pl.BlockSpec((1, H, H), lambda g, be: (be[g], 0, 0)),
                pl.BlockSpec((1, 1, H), lambda g, be: (be[g], 0, 0)),
            ],
            out_specs=pl.BlockSpec((BLK, H), lambda g, be: (g, 0)),
        ),
        out_shape=jax.ShapeDtypeStruct((XR, H), jnp.float32),
    )(block_expert, xg, We0, be0[:, None, :], We1, be1[:, None, :])

    pos2 = pos.reshape(T, B, K)
    pos_cat = jnp.concatenate(
        [pos2[:, :, 0].reshape(-1), pos2[:, :, 1].reshape(-1)])
    rows_cat = _sc_gather(out_rows, pos_cat, 2 * T * B, 64)
    rc = rows_cat.reshape(2, T, B, H)

    HB = 2048
    out = pl.pallas_call(
        _heads_body,
        grid=(T, B // HB),
        in_specs=[
            pl.BlockSpec((2, 1, HB, H), lambda t, i: (0, t, i, 0)),
            pl.BlockSpec((1, HB, 1), lambda t, i: (t, i, 0)),
            pl.BlockSpec((1, HB, 1), lambda t, i: (t, i, 0)),
            pl.BlockSpec((1, H, 512), lambda t, i: (t, 0, 0)),
            pl.BlockSpec((1, 1, 512), lambda t, i: (t, 0, 0)),
            pl.BlockSpec((1, 512, 256), lambda t, i: (t, 0, 0)),
            pl.BlockSpec((1, 1, 256), lambda t, i: (t, 0, 0)),
            pl.BlockSpec((1, 256, 1), lambda t, i: (t, 0, 0)),
            pl.BlockSpec((1, 1, 1), lambda t, i: (t, 0, 0)),
        ],
        out_specs=pl.BlockSpec((1, HB, 1), lambda t, i: (t, i, 0)),
        out_shape=jax.ShapeDtypeStruct((T, B, 1), jnp.float32),
    )(rc, meta[:, :, 2:3], meta[:, :, 3:4], Wh0, bh0[:, None, :], Wh1,
      bh1[:, None, :], Wh2, bh2[:, None, :])
    return out


# speedup vs baseline: 2.0321x; 2.0321x over previous
import functools

import jax
import jax.numpy as jnp
from jax.experimental import pallas as pl
from jax.experimental.pallas import tpu as pltpu

B = 4096
D = 2048
E = 16
H = 512
T = 2
TOPK = 2

_GATE_BB = 1024
_EXP_BB = 2048


def _gate_body(x_ref, wcat_ref, eps_ref, g_ref, xbf_ref):
    x = x_ref[...]
    xbf_ref[...] = x.astype(jnp.bfloat16)
    proj = jnp.dot(x, wcat_ref[...], preferred_element_type=jnp.float32)
    ii = jax.lax.broadcasted_iota(jnp.int32, (x.shape[0], E), 1)
    for t in range(T):
        mean = proj[:, t * E:(t + 1) * E]
        npj = proj[:, (T + t) * E:(T + t + 1) * E]
        std = jnp.maximum(npj, 0.0) + jnp.log1p(jnp.exp(-jnp.abs(npj)))
        noisy = mean + eps_ref[t] * std
        v1 = jnp.max(noisy, axis=1, keepdims=True)
        first1 = jnp.min(jnp.where(noisy == v1, ii, E), axis=1, keepdims=True)
        n2 = jnp.where(ii == first1, -jnp.inf, noisy)
        v2 = jnp.max(n2, axis=1, keepdims=True)
        routing = jnp.where(noisy < v2, -jnp.float32(1e30), noisy)
        ex = jnp.exp(routing - v1)
        g_ref[t] = ex / jnp.sum(ex, axis=1, keepdims=True)


def _expert_body(x_ref, we0_ref, be0_ref, we1_ref, be1_ref, g_ref, go_ref):
    e = pl.program_id(1)

    @pl.when(e == 0)
    def _():
        go_ref[...] = jnp.zeros_like(go_ref)

    h = jnp.maximum(
        jnp.dot(x_ref[...], we0_ref[0], preferred_element_type=jnp.float32)
        + be0_ref[0], 0.0).astype(jnp.bfloat16)
    o = jnp.dot(h, we1_ref[0], preferred_element_type=jnp.float32) \
        + be1_ref[0]
    lane = jax.lax.broadcasted_iota(jnp.int32, (x_ref.shape[0], E), 1)
    for t in range(T):
        gcol = jnp.sum(jnp.where(lane == e, g_ref[t], 0.0), axis=1,
                       keepdims=True)
        go_ref[t] += gcol * o


def _head_body(go_ref, wh0_ref, bh0_ref, wh1_ref, bh1_ref, wh2_ref, bh2_ref,
               out_ref):
    a = jnp.maximum(
        jnp.dot(go_ref[0], wh0_ref[0], preferred_element_type=jnp.float32)
        + bh0_ref[0], 0.0)
    b = jnp.maximum(
        jnp.dot(a, wh1_ref[0], preferred_element_type=jnp.float32)
        + bh1_ref[0], 0.0)
    out_ref[0] = jnp.dot(b, wh2_ref[0], preferred_element_type=jnp.float32) \
        + bh2_ref[0]


def kernel(x, We0, be0, We1, be1, Wg, Wn, Wh0, bh0, Wh1, bh1, Wh2, bh2):
    eps_key = jax.random.key(42)
    eps = jnp.stack([
        jax.random.normal(jax.random.fold_in(eps_key, i), (B, E), jnp.float32)
        for i in range(T)])

    wcat = jnp.concatenate(
        [Wg[0], Wg[1], Wn[0], Wn[1]], axis=1)

    g = pl.pallas_call(
        _gate_body,
        grid=(B // _GATE_BB,),
        in_specs=[
            pl.BlockSpec((_GATE_BB, D), lambda i: (i, 0)),
            pl.BlockSpec((D, 4 * E), lambda i: (0, 0)),
            pl.BlockSpec((T, _GATE_BB, E), lambda i: (0, i, 0)),
        ],
        out_specs=[
            pl.BlockSpec((T, _GATE_BB, E), lambda i: (0, i, 0)),
            pl.BlockSpec((_GATE_BB, D), lambda i: (i, 0)),
        ],
        out_shape=[
            jax.ShapeDtypeStruct((T, B, E), jnp.float32),
            jax.ShapeDtypeStruct((B, D), jnp.bfloat16),
        ],
    )(x, wcat, eps)
    g, x_bf = g

    go = pl.pallas_call(
        _expert_body,
        grid=(B // _EXP_BB, E),
        in_specs=[
            pl.BlockSpec((_EXP_BB, D), lambda i, e: (i, 0)),
            pl.BlockSpec((1, D, H), lambda i, e: (e, 0, 0)),
            pl.BlockSpec((1, 1, H), lambda i, e: (e, 0, 0)),
            pl.BlockSpec((1, H, H), lambda i, e: (e, 0, 0)),
            pl.BlockSpec((1, 1, H), lambda i, e: (e, 0, 0)),
            pl.BlockSpec((T, _EXP_BB, E), lambda i, e: (0, i, 0)),
        ],
        out_specs=pl.BlockSpec((T, _EXP_BB, H), lambda i, e: (0, i, 0)),
        out_shape=jax.ShapeDtypeStruct((T, B, H), jnp.float32),
    )(x_bf, We0.astype(jnp.bfloat16), be0[:, None, :],
      We1.astype(jnp.bfloat16), be1[:, None, :], g)

    out = pl.pallas_call(
        _head_body,
        grid=(T,),
        in_specs=[
            pl.BlockSpec((1, B, H), lambda t: (t, 0, 0)),
            pl.BlockSpec((1, H, 512), lambda t: (t, 0, 0)),
            pl.BlockSpec((1, 1, 512), lambda t: (t, 0, 0)),
            pl.BlockSpec((1, 512, 256), lambda t: (t, 0, 0)),
            pl.BlockSpec((1, 1, 256), lambda t: (t, 0, 0)),
            pl.BlockSpec((1, 256, 1), lambda t: (t, 0, 0)),
            pl.BlockSpec((1, 1, 1), lambda t: (t, 0, 0)),
        ],
        out_specs=pl.BlockSpec((1, B, 1), lambda t: (t, 0, 0)),
        out_shape=jax.ShapeDtypeStruct((T, B, 1), jnp.float32),
    )(go, Wh0, bh0[:, None, :], Wh1, bh1[:, None, :], Wh2, bh2[:, None, :])
    return out


# speedup vs baseline: 2.2294x; 1.0971x over previous
import functools

import jax
import jax.numpy as jnp
from jax.experimental import pallas as pl
from jax.experimental.pallas import tpu as pltpu

B = 4096
D = 2048
E = 16
H = 512
T = 2
TOPK = 2

_GATE_BB = 1024
_EXP_BB = 1024


def _gate_body(x_ref, wcat_ref, eps_ref, g_ref):
    x = x_ref[...]
    proj = jnp.dot(x, wcat_ref[...], preferred_element_type=jnp.float32)
    ii = jax.lax.broadcasted_iota(jnp.int32, (x.shape[0], E), 1)
    for t in range(T):
        mean = proj[:, t * E:(t + 1) * E]
        npj = proj[:, (T + t) * E:(T + t + 1) * E]
        std = jnp.maximum(npj, 0.0) + jnp.log1p(jnp.exp(-jnp.abs(npj)))
        noisy = mean + eps_ref[t] * std
        v1 = jnp.max(noisy, axis=1, keepdims=True)
        first1 = jnp.min(jnp.where(noisy == v1, ii, E), axis=1, keepdims=True)
        n2 = jnp.where(ii == first1, -jnp.inf, noisy)
        v2 = jnp.max(n2, axis=1, keepdims=True)
        routing = jnp.where(noisy < v2, -jnp.float32(1e30), noisy)
        ex = jnp.exp(routing - v1)
        g_ref[t] = ex / jnp.sum(ex, axis=1, keepdims=True)


def _expert_body(x_ref, we0_ref, be0_ref, we1_ref, be1_ref, g_ref, go_ref):
    e = pl.program_id(1)

    @pl.when(e == 0)
    def _():
        go_ref[...] = jnp.zeros_like(go_ref)

    h = jnp.maximum(
        jnp.dot(x_ref[...], we0_ref[0], preferred_element_type=jnp.float32)
        + be0_ref[0], 0.0)
    o = jnp.dot(h, we1_ref[0], preferred_element_type=jnp.float32) \
        + be1_ref[0]
    lane = jax.lax.broadcasted_iota(jnp.int32, (x_ref.shape[0], E), 1)
    for t in range(T):
        gcol = jnp.sum(jnp.where(lane == e, g_ref[t], 0.0), axis=1,
                       keepdims=True)
        go_ref[t] += gcol * o


def _head_body(go_ref, wh0_ref, bh0_ref, wh1_ref, bh1_ref, wh2_ref, bh2_ref,
               out_ref):
    a = jnp.maximum(
        jnp.dot(go_ref[0], wh0_ref[0], preferred_element_type=jnp.float32)
        + bh0_ref[0], 0.0)
    b = jnp.maximum(
        jnp.dot(a, wh1_ref[0], preferred_element_type=jnp.float32)
        + bh1_ref[0], 0.0)
    out_ref[0] = jnp.dot(b, wh2_ref[0], preferred_element_type=jnp.float32) \
        + bh2_ref[0]


def kernel(x, We0, be0, We1, be1, Wg, Wn, Wh0, bh0, Wh1, bh1, Wh2, bh2):
    eps_key = jax.random.key(42)
    eps = jnp.stack([
        jax.random.normal(jax.random.fold_in(eps_key, i), (B, E), jnp.float32)
        for i in range(T)])

    wcat = jnp.concatenate(
        [Wg[0], Wg[1], Wn[0], Wn[1]], axis=1)

    g = pl.pallas_call(
        _gate_body,
        grid=(B // _GATE_BB,),
        in_specs=[
            pl.BlockSpec((_GATE_BB, D), lambda i: (i, 0)),
            pl.BlockSpec((D, 4 * E), lambda i: (0, 0)),
            pl.BlockSpec((T, _GATE_BB, E), lambda i: (0, i, 0)),
        ],
        out_specs=pl.BlockSpec((T, _GATE_BB, E), lambda i: (0, i, 0)),
        out_shape=jax.ShapeDtypeStruct((T, B, E), jnp.float32),
    )(x, wcat, eps)

    go = pl.pallas_call(
        _expert_body,
        grid=(B // _EXP_BB, E),
        in_specs=[
            pl.BlockSpec((_EXP_BB, D), lambda i, e: (i, 0)),
            pl.BlockSpec((1, D, H), lambda i, e: (e, 0, 0)),
            pl.BlockSpec((1, 1, H), lambda i, e: (e, 0, 0)),
            pl.BlockSpec((1, H, H), lambda i, e: (e, 0, 0)),
            pl.BlockSpec((1, 1, H), lambda i, e: (e, 0, 0)),
            pl.BlockSpec((T, _EXP_BB, E), lambda i, e: (0, i, 0)),
        ],
        out_specs=pl.BlockSpec((T, _EXP_BB, H), lambda i, e: (0, i, 0)),
        out_shape=jax.ShapeDtypeStruct((T, B, H), jnp.float32),
    )(x, We0, be0[:, None, :], We1, be1[:, None, :], g)

    out = pl.pallas_call(
        _head_body,
        grid=(T,),
        in_specs=[
            pl.BlockSpec((1, B, H), lambda t: (t, 0, 0)),
            pl.BlockSpec((1, H, 512), lambda t: (t, 0, 0)),
            pl.BlockSpec((1, 1, 512), lambda t: (t, 0, 0)),
            pl.BlockSpec((1, 512, 256), lambda t: (t, 0, 0)),
            pl.BlockSpec((1, 1, 256), lambda t: (t, 0, 0)),
            pl.BlockSpec((1, 256, 1), lambda t: (t, 0, 0)),
            pl.BlockSpec((1, 1, 1), lambda t: (t, 0, 0)),
        ],
        out_specs=pl.BlockSpec((1, B, 1), lambda t: (t, 0, 0)),
        out_shape=jax.ShapeDtypeStruct((T, B, 1), jnp.float32),
    )(go, Wh0, bh0[:, None, :], Wh1, bh1[:, None, :], Wh2, bh2[:, None, :])
    return out
